# baseline (speedup 1.0000x reference)
import functools

import jax
import jax.numpy as jnp
from jax import lax
from jax.experimental import pallas as pl
from jax.experimental.pallas import tpu as pltpu
from jax.experimental.pallas import tpu_sc as plsc

BATCH = 16384
EMB_DIM = 64
NC = 2
NS = 16
NW = NC * NS
B_PER_W = BATCH // NW
GCHUNK = 128
NCHUNK = B_PER_W // GCHUNK

SLOPE = 0.01
OUT_PAD = 128


def _sc_gather_body(idx_hbm, table_hbm, out_hbm, idx_v, rows_v, sem):
    wid = lax.axis_index("s") * NC + lax.axis_index("c")
    base = wid * B_PER_W
    pltpu.sync_copy(idx_hbm.at[wid], idx_v)
    copies = []
    for c in range(NCHUNK):
        copies.append(pltpu.async_copy(
            table_hbm.at[idx_v.at[c]],
            rows_v.at[pl.ds(c * GCHUNK, GCHUNK)],
            sem,
        ))
    for cp in copies:
        cp.wait()
    pltpu.sync_copy(rows_v, out_hbm.at[pl.ds(base, B_PER_W)])


@functools.partial(jax.jit, static_argnames=())
def _sc_gather(user, emb):
    mesh = plsc.VectorSubcoreMesh(core_axis_name="c", subcore_axis_name="s")
    k = functools.partial(
        pl.kernel,
        mesh=mesh,
        out_type=jax.ShapeDtypeStruct((BATCH, EMB_DIM), jnp.float32),
        scratch_types=[
            pltpu.VMEM((NCHUNK, GCHUNK), jnp.int32),
            pltpu.VMEM((B_PER_W, EMB_DIM), jnp.float32),
            pltpu.SemaphoreType.DMA,
        ],
        compiler_params=pltpu.CompilerParams(use_tc_tiling_on_sc=False),
    )(_sc_gather_body)
    idx3 = user.reshape(NW, NCHUNK, GCHUNK)
    return k(idx3, emb)


def _mlp_body(g_ref, s_ref, w1e_ref, w1s_ref, b1_ref, w2_ref, b2_ref,
              w3_ref, b3_ref, w4_ref, b4_ref, o_ref):
    h = jnp.dot(g_ref[...], w1e_ref[...], preferred_element_type=jnp.float32)
    h = h + jnp.dot(s_ref[...], w1s_ref[...], preferred_element_type=jnp.float32)
    h = h + b1_ref[...]
    h = jnp.where(h >= 0, h, SLOPE * h)
    h = jnp.dot(h, w2_ref[...], preferred_element_type=jnp.float32) + b2_ref[...]
    h = jnp.where(h >= 0, h, SLOPE * h)
    h = jnp.dot(h, w3_ref[...], preferred_element_type=jnp.float32) + b3_ref[...]
    h = jnp.where(h >= 0, h, SLOPE * h)
    o_ref[...] = jnp.dot(h, w4_ref[...], preferred_element_type=jnp.float32) + b4_ref[...]


MLP_BLK = 2048


def _mlp(gathered, score, w1e, w1s, b1, w2, b2, w3, b3, w4, b4):
    grid = (BATCH // MLP_BLK,)
    const = lambda i: (0, 0)
    return pl.pallas_call(
        _mlp_body,
        grid=grid,
        in_specs=[
            pl.BlockSpec((MLP_BLK, EMB_DIM), lambda i: (i, 0)),
            pl.BlockSpec((MLP_BLK, EMB_DIM), lambda i: (i, 0)),
            pl.BlockSpec(w1e.shape, const),
            pl.BlockSpec(w1s.shape, const),
            pl.BlockSpec(b1.shape, const),
            pl.BlockSpec(w2.shape, const),
            pl.BlockSpec(b2.shape, const),
            pl.BlockSpec(w3.shape, const),
            pl.BlockSpec(b3.shape, const),
            pl.BlockSpec(w4.shape, const),
            pl.BlockSpec(b4.shape, const),
        ],
        out_specs=pl.BlockSpec((MLP_BLK, OUT_PAD), lambda i: (i, 0)),
        out_shape=jax.ShapeDtypeStruct((BATCH, OUT_PAD), jnp.float32),
        compiler_params=pltpu.CompilerParams(
            dimension_semantics=("arbitrary",),
        ),
    )(gathered, score, w1e, w1s, b1, w2, b2, w3, b3, w4, b4)


def kernel(user, score, emb, W1, b1, W2, b2, W3, b3, W4, b4):
    gathered = _sc_gather(user, emb)
    w1t = W1.T
    w1e = w1t[:EMB_DIM]
    w1s = w1t[EMB_DIM:]
    w2t = W2.T
    w3t = W3.T
    w4t = jnp.pad(W4.T, ((0, 0), (0, OUT_PAD - 5)))
    b4p = jnp.pad(b4, (0, OUT_PAD - 5))
    out = _mlp(gathered, score,
               w1e, w1s, b1.reshape(1, -1),
               w2t, b2.reshape(1, -1),
               w3t, b3.reshape(1, -1),
               w4t, b4p.reshape(1, -1))
    return out[:, :5]
